# baseline (speedup 1.0000x reference)
import functools

import jax
import jax.numpy as jnp
from jax import lax
from jax.experimental import pallas as pl
from jax.experimental.pallas import tpu as pltpu
from jax.experimental.pallas import tpu_sc as plsc

VOCAB = 3000000
DIM = 300
LANES = 16
NC, NS = 2, 16
VB = 8192
GRID = -(-VOCAB // VB)
CLEN = GRID * VB
REG = CLEN // 4
TRASH = 128
SCHUNK = 128
ZLEN = 8192
CSTAGE = 16384
N_IDX = 1024 * 200
NPT = N_IDX // NS


def _counts_call():
    mesh = plsc.VectorSubcoreMesh(
        core_axis_name="c", subcore_axis_name="s",
        num_cores=NC, num_subcores=NS)
    zchunks, zrem = divmod(REG // NS, ZLEN)
    wchunks, wrem = divmod(REG // NS, CSTAGE)

    @functools.partial(
        pl.kernel,
        out_type=jax.ShapeDtypeStruct((CLEN,), jnp.float32),
        mesh=mesh,
        scratch_types=[
            pltpu.VMEM((NPT,), jnp.int32),
            pltpu.VMEM((NPT // 128, 128), jnp.int32),
            pltpu.VMEM((SCHUNK,), jnp.float32),
            pltpu.VMEM((ZLEN,), jnp.float32),
            pltpu.VMEM((CSTAGE,), jnp.float32),
            pltpu.VMEM_SHARED((REG + TRASH,), jnp.float32),
        ],
    )
    def counts_sc(xf, zeros_h, ones_h, c_out, idx_v, sidx_v, ones_v,
                  zbuf, cstage, csh):
        cid = lax.axis_index("c")
        sid = lax.axis_index("s")
        pltpu.sync_copy(xf.at[pl.ds(sid * NPT, NPT)], idx_v)
        pltpu.sync_copy(ones_h, ones_v)
        pltpu.sync_copy(zeros_h, zbuf)
        lane = lax.iota(jnp.int32, LANES)

        for p in range(2):
            base_v = (p * NC + cid) * REG

            zoff = sid * (REG // NS)
            for j in range(zchunks):
                pltpu.sync_copy(zbuf, csh.at[pl.ds(zoff + j * ZLEN, ZLEN)])
            if zrem:
                pltpu.sync_copy(
                    zbuf.at[pl.ds(0, zrem)],
                    csh.at[pl.ds(zoff + zchunks * ZLEN, zrem)])

            @pl.when(sid == 0)
            def _():
                pltpu.sync_copy(zbuf.at[pl.ds(0, TRASH)],
                                csh.at[pl.ds(REG, TRASH)])

            def remap(i, carry):
                v = idx_v[pl.ds(i * LANES, LANES)]
                inr = jnp.logical_and(v >= base_v, v < base_v + REG)
                trash = REG + ((i * LANES + lane) & (TRASH - 1))
                sp = jnp.where(inr, v - base_v, trash)
                sidx_v[i // 8, pl.ds((i % 8) * LANES, LANES)] = sp
                return carry

            lax.fori_loop(0, NPT // LANES, remap, 0)
            plsc.subcore_barrier()

            def scat(j, carry):
                pltpu.sync_copy(ones_v, csh.at[sidx_v.at[j]], add=True)
                return carry

            lax.fori_loop(0, NPT // SCHUNK, scat, 0)
            plsc.subcore_barrier()

            for j in range(wchunks):
                off = sid * (REG // NS) + j * CSTAGE
                pltpu.sync_copy(csh.at[pl.ds(off, CSTAGE)], cstage)
                pltpu.sync_copy(cstage,
                                c_out.at[pl.ds(base_v + off, CSTAGE)])
            if wrem:
                off = sid * (REG // NS) + wchunks * CSTAGE
                pltpu.sync_copy(csh.at[pl.ds(off, wrem)],
                                cstage.at[pl.ds(0, wrem)])
                pltpu.sync_copy(cstage.at[pl.ds(0, wrem)],
                                c_out.at[pl.ds(base_v + off, wrem)])
            if p == 0:
                plsc.subcore_barrier()

    return counts_sc


def _matvec(tt, c1):
    nch = VB // 128

    def chunk_sum(t, cb):
        con = t[:, 0:128] * jnp.broadcast_to(cb[0:1, :], (DIM, 128))
        for k in range(1, nch):
            con += (t[:, k * 128:(k + 1) * 128]
                    * jnp.broadcast_to(cb[k:k + 1, :], (DIM, 128)))
        return con

    def body(t_ref, c_ref, out_ref):
        g = pl.program_id(0)

        @pl.when(g == 0)
        def _():
            out_ref[...] = jnp.zeros_like(out_ref)

        cb = c_ref[...].reshape(nch, 128)

        @pl.when(g < GRID - 1)
        def _():
            out_ref[...] += chunk_sum(t_ref[...], cb)

        @pl.when(g == GRID - 1)
        def _():
            valid = (lax.broadcasted_iota(jnp.int32, (DIM, VB), 1)
                     < VOCAB - (GRID - 1) * VB)
            out_ref[...] += chunk_sum(
                jnp.where(valid, t_ref[...], 0.0), cb)

    return pl.pallas_call(
        body,
        grid=(GRID,),
        in_specs=[
            pl.BlockSpec((DIM, VB), lambda g: (0, g)),
            pl.BlockSpec((VB,), lambda g: (g,)),
        ],
        out_specs=pl.BlockSpec((DIM, 128), lambda g: (0, 0)),
        out_shape=jax.ShapeDtypeStruct((DIM, 128), jnp.float32),
    )(tt, c1)


def _probe_call(niter):
    mesh = plsc.VectorSubcoreMesh(
        core_axis_name="c", subcore_axis_name="s",
        num_cores=NC, num_subcores=NS)
    cw = 4096

    @functools.partial(
        pl.kernel,
        out_type=jax.ShapeDtypeStruct((NC * NS, LANES), jnp.float32),
        mesh=mesh,
        scratch_types=[
            pltpu.VMEM((8, cw), jnp.float32),
            pltpu.SemaphoreType.DMA,
        ],
    )
    def probe(tt, out, buf, sem):
        wid = lax.axis_index("s") * NC + lax.axis_index("c")
        colbase = wid * 89600

        def it(i, carry):
            rb = 8 * (i % 37)
            cb = colbase + (i // 37) * cw
            pltpu.async_copy(tt.at[pl.ds(rb, 8), pl.ds(cb, cw)], buf,
                             sem).wait()
            return carry + buf[0, pl.ds(0, LANES)]

        acc = lax.fori_loop(0, niter, it,
                            jnp.zeros((LANES,), jnp.float32))
        buf[0, pl.ds(0, LANES)] = acc
        pltpu.sync_copy(buf.at[0, pl.ds(0, LANES)], out.at[wid])

    return probe


def kernel(x, table, W, b):
    xf = x.reshape(-1).astype(jnp.int32)
    zeros_h = jnp.zeros((ZLEN,), jnp.float32)
    ones_h = jnp.ones((SCHUNK,), jnp.float32)
    c1 = _counts_call()(xf, zeros_h, ones_h)
    acc = _matvec(table.T, c1)
    pr = _probe_call(128)(table.T)
    s = jnp.sum(acc, axis=1)
    wsum = W[:, 0] + W[:, 1]
    return (jnp.dot(s, wsum) / (x.size * 2) + jnp.mean(b)
            + 0.0 * pr[0, 0])


# speedup vs baseline: 1.1626x; 1.1626x over previous
import functools

import jax
import jax.numpy as jnp
from jax import lax
from jax.experimental import pallas as pl
from jax.experimental.pallas import tpu as pltpu
from jax.experimental.pallas import tpu_sc as plsc

VOCAB = 3000000
DIM = 300
LANES = 16
NC, NS = 2, 16
VB = 8192
GRID = 368
GRIDH = GRID // 2
CLEN = GRID * VB
REG = CLEN // 4
TRASH = 128
SCHUNK = 128
ZLEN = 8192
CSTAGE = 16384
N_IDX = 1024 * 200
NPT = N_IDX // NS


def _counts_call(p):
    mesh = plsc.VectorSubcoreMesh(
        core_axis_name="c", subcore_axis_name="s",
        num_cores=NC, num_subcores=NS)
    zchunks, zrem = divmod(REG // NS, ZLEN)
    wchunks, wrem = divmod(REG // NS, CSTAGE)

    @functools.partial(
        pl.kernel,
        out_type=jax.ShapeDtypeStruct((2 * REG,), jnp.float32),
        mesh=mesh,
        scratch_types=[
            pltpu.VMEM((NPT,), jnp.int32),
            pltpu.VMEM((NPT // 128, 128), jnp.int32),
            pltpu.VMEM((SCHUNK,), jnp.float32),
            pltpu.VMEM((ZLEN,), jnp.float32),
            pltpu.VMEM((CSTAGE,), jnp.float32),
            pltpu.VMEM_SHARED((REG + TRASH,), jnp.float32),
        ],
    )
    def counts_sc(xf, zeros_h, ones_h, c_out, idx_v, sidx_v, ones_v,
                  zbuf, cstage, csh):
        cid = lax.axis_index("c")
        sid = lax.axis_index("s")
        pltpu.sync_copy(xf.at[pl.ds(sid * NPT, NPT)], idx_v)
        pltpu.sync_copy(ones_h, ones_v)
        pltpu.sync_copy(zeros_h, zbuf)
        lane = lax.iota(jnp.int32, LANES)
        base_v = (p * NC + cid) * REG
        obase = cid * REG

        zoff = sid * (REG // NS)
        for j in range(zchunks):
            pltpu.sync_copy(zbuf, csh.at[pl.ds(zoff + j * ZLEN, ZLEN)])
        if zrem:
            pltpu.sync_copy(
                zbuf.at[pl.ds(0, zrem)],
                csh.at[pl.ds(zoff + zchunks * ZLEN, zrem)])

        @pl.when(sid == 0)
        def _():
            pltpu.sync_copy(zbuf.at[pl.ds(0, TRASH)],
                            csh.at[pl.ds(REG, TRASH)])

        def remap(i, carry):
            v = idx_v[pl.ds(i * LANES, LANES)]
            inr = jnp.logical_and(v >= base_v, v < base_v + REG)
            trash = REG + ((i * LANES + lane) & (TRASH - 1))
            sp = jnp.where(inr, v - base_v, trash)
            sidx_v[i // 8, pl.ds((i % 8) * LANES, LANES)] = sp
            return carry

        lax.fori_loop(0, NPT // LANES, remap, 0)
        plsc.subcore_barrier()

        def scat(j, carry):
            pltpu.sync_copy(ones_v, csh.at[sidx_v.at[j]], add=True)
            return carry

        lax.fori_loop(0, NPT // SCHUNK, scat, 0)
        plsc.subcore_barrier()

        for j in range(wchunks):
            off = sid * (REG // NS) + j * CSTAGE
            pltpu.sync_copy(csh.at[pl.ds(off, CSTAGE)], cstage)
            pltpu.sync_copy(cstage, c_out.at[pl.ds(obase + off, CSTAGE)])
        if wrem:
            off = sid * (REG // NS) + wchunks * CSTAGE
            pltpu.sync_copy(csh.at[pl.ds(off, wrem)],
                            cstage.at[pl.ds(0, wrem)])
            pltpu.sync_copy(cstage.at[pl.ds(0, wrem)],
                            c_out.at[pl.ds(obase + off, wrem)])

    return counts_sc


def _matvec(tt, ch, h):
    nch = VB // 128
    ngrid = GRIDH if h == 0 else GRIDH - 1
    gfull = (VOCAB // VB) - h * GRIDH

    def chunk_sum(t, cb):
        con = t[:, 0:128] * jnp.broadcast_to(cb[0:1, :], (DIM, 128))
        for k in range(1, nch):
            con += (t[:, k * 128:(k + 1) * 128]
                    * jnp.broadcast_to(cb[k:k + 1, :], (DIM, 128)))
        return con

    def body(t_ref, c_ref, out_ref):
        g = pl.program_id(0)

        @pl.when(g == 0)
        def _():
            out_ref[...] = jnp.zeros_like(out_ref)

        cb = c_ref[...].reshape(nch, 128)
        if h == 0:
            out_ref[...] += chunk_sum(t_ref[...], cb)
        else:
            @pl.when(g < gfull)
            def _():
                out_ref[...] += chunk_sum(t_ref[...], cb)

            @pl.when(g >= gfull)
            def _():
                valid = (lax.broadcasted_iota(jnp.int32, (DIM, VB), 1)
                         < VOCAB - (h * GRIDH + g) * VB)
                out_ref[...] += chunk_sum(
                    jnp.where(valid, t_ref[...], 0.0), cb)

    return pl.pallas_call(
        body,
        grid=(ngrid,),
        in_specs=[
            pl.BlockSpec((DIM, VB), lambda g: (0, h * GRIDH + g)),
            pl.BlockSpec((VB,), lambda g: (g,)),
        ],
        out_specs=pl.BlockSpec((DIM, 128), lambda g: (0, 0)),
        out_shape=jax.ShapeDtypeStruct((DIM, 128), jnp.float32),
    )(tt, ch)


def kernel(x, table, W, b):
    xf = x.reshape(-1).astype(jnp.int32)
    zeros_h = jnp.zeros((ZLEN,), jnp.float32)
    ones_h = jnp.ones((SCHUNK,), jnp.float32)
    tt = table.T
    c_lo = _counts_call(0)(xf, zeros_h, ones_h)
    c_hi = _counts_call(1)(xf, zeros_h, ones_h)
    acc = _matvec(tt, c_lo, 0) + _matvec(tt, c_hi, 1)
    s = jnp.sum(acc, axis=1)
    wsum = W[:, 0] + W[:, 1]
    return jnp.dot(s, wsum) / (x.size * 2) + jnp.mean(b)
